# baseline (speedup 1.0000x reference)
import functools

import jax
import jax.numpy as jnp
from jax import lax
from jax.experimental import pallas as pl
from jax.experimental.pallas import tpu as pltpu
from jax.experimental.pallas import tpu_sc as plsc

D_MODEL = 768
D_FF = 2048
N_EXP = 64
BLK = 128
NW = 32

def _sc_mesh():
    return plsc.VectorSubcoreMesh(core_axis_name="c", subcore_axis_name="s")


def _route_body(x_ref, gw_ref, gb_ref, eids_ref, nact_ref, tok2_ref, w2_ref,
                p0_ref, p1_ref):
    S = x_ref.shape[0]
    NB = 2 * S // BLK + N_EXP
    CH = 128
    NCH = 2 * S // CH
    HALF = S // CH

    s = jax.lax.dot_general(x_ref[...], gw_ref[...], (((1,), (1,)), ((), ())),
                            precision=jax.lax.Precision.HIGHEST,
                            preferred_element_type=jnp.float32)
    s = s + gb_ref[...]
    col = jax.lax.broadcasted_iota(jnp.int32, s.shape, 1)
    m0 = jnp.max(s, axis=1, keepdims=True)
    i0 = jnp.min(jnp.where(s == m0, col, N_EXP), axis=1, keepdims=True)
    s1 = jnp.where(col == i0, -jnp.inf, s)
    m1 = jnp.max(s1, axis=1, keepdims=True)
    i1 = jnp.min(jnp.where(s1 == m1, col, N_EXP), axis=1, keepdims=True)
    z = jnp.exp(m1 - m0)
    w0 = 1.0 / (1.0 + z)
    w1 = z / (1.0 + z)

    e64 = jax.lax.broadcasted_iota(jnp.int32, (CH, N_EXP), 1)
    lt = (jax.lax.broadcasted_iota(jnp.int32, (CH, CH), 1)
          < jax.lax.broadcasted_iota(jnp.int32, (CH, CH), 0)).astype(
              jnp.float32)

    def pair_chunk(c):
        if c < HALF:
            return (jax.lax.slice(i0, (c * CH, 0), ((c + 1) * CH, 1)),
                    jax.lax.slice(w0, (c * CH, 0), ((c + 1) * CH, 1)))
        cc = c - HALF
        return (jax.lax.slice(i1, (cc * CH, 0), ((cc + 1) * CH, 1)),
                jax.lax.slice(w1, (cc * CH, 0), ((cc + 1) * CH, 1)))

    running = jnp.zeros((1, N_EXP), jnp.float32)
    ranks = []
    for c in range(NCH):
        ec, _ = pair_chunk(c)
        oh = (ec == e64).astype(jnp.float32)
        base = jnp.sum(running * oh, axis=1, keepdims=True)
        rwithin = jax.lax.dot_general(lt, oh, (((1,), (0,)), ((), ())),
                                      preferred_element_type=jnp.float32)
        rank_in = jnp.sum(rwithin * oh, axis=1, keepdims=True)
        ranks.append(base + rank_in)
        running = running + jnp.sum(oh, axis=0, keepdims=True)
    counts = running

    m_incl = (jax.lax.broadcasted_iota(jnp.int32, (N_EXP, N_EXP), 0)
              <= jax.lax.broadcasted_iota(jnp.int32, (N_EXP, N_EXP), 1)
              ).astype(jnp.float32)
    nblk = jnp.floor((counts + (BLK - 1)) * (1.0 / BLK))
    cumb = jax.lax.dot_general(nblk, m_incl, (((1,), (0,)), ((), ())),
                               preferred_element_type=jnp.float32)
    padded_off = (cumb - nblk) * BLK
    nact_ref[...] = jax.lax.slice(cumb, (0, N_EXP - 1),
                                  (1, N_EXP)).astype(jnp.int32)

    ident = (jax.lax.broadcasted_iota(jnp.int32, (N_EXP, N_EXP), 0)
             == jax.lax.broadcasted_iota(jnp.int32, (N_EXP, N_EXP), 1)
             ).astype(jnp.float32)
    cumb_col = jax.lax.dot_general(ident, cumb, (((1,), (1,)), ((), ())),
                                   preferred_element_type=jnp.float32)
    b_row = jax.lax.broadcasted_iota(jnp.int32, (1, NB), 1).astype(jnp.float32)
    eids = jnp.sum((cumb_col <= b_row).astype(jnp.float32), axis=0,
                   keepdims=True)
    e_iota = jax.lax.broadcasted_iota(jnp.int32, (1, N_EXP),
                                      1).astype(jnp.float32)
    last_e = jnp.max(jnp.where(counts > 0.0, e_iota, 0.0))
    eids_ref[...] = jnp.minimum(eids, last_e).astype(jnp.int32)

    sub128 = jax.lax.broadcasted_iota(jnp.int32, (CH, 1),
                                      0).astype(jnp.float32)
    b_cols = jax.lax.broadcasted_iota(jnp.int32, (CH, NB), 1)
    r_cols = jax.lax.broadcasted_iota(jnp.int32, (CH, CH), 1)
    acc_t = jnp.zeros((NB, BLK), jnp.float32)
    acc_w = jnp.zeros((NB, BLK), jnp.float32)
    for c in range(NCH):
        ec, wc = pair_chunk(c)
        oh = (ec == e64).astype(jnp.float32)
        pos = jnp.sum(padded_off * oh, axis=1, keepdims=True) + ranks[c]
        posi = pos.astype(jnp.int32)
        if c < HALF:
            p0_ref[pl.ds(c * CH, CH), :] = posi
        else:
            p1_ref[pl.ds((c - HALF) * CH, CH), :] = posi
        pos_b = jnp.floor(pos * (1.0 / BLK))
        pos_r = pos - pos_b * BLK
        rsel = (pos_b.astype(jnp.int32) == b_cols).astype(jnp.float32)
        csel = (pos_r.astype(jnp.int32) == r_cols).astype(jnp.float32)
        tokv = (c % HALF) * CH + sub128
        acc_t = acc_t + jax.lax.dot_general(
            rsel * tokv, csel, (((0,), (0,)), ((), ())),
            precision=jax.lax.Precision.HIGHEST,
            preferred_element_type=jnp.float32)
        acc_w = acc_w + jax.lax.dot_general(
            rsel * wc, csel, (((0,), (0,)), ((), ())),
            precision=jax.lax.Precision.HIGHEST,
            preferred_element_type=jnp.float32)
    tok2_ref[...] = acc_t.astype(jnp.int32)
    w2_ref[...] = acc_w


def _ffn_body(eids_ref, nact_ref, x_ref, tok_ref, w1_ref, b1_ref, w2_ref,
              b2_ref, g_ref, be_ref, wp_ref, o_ref):
    @pl.when(pl.program_id(0) < nact_ref[0])
    def _():
        tok = tok_ref[...]
        sel = jax.lax.broadcasted_iota(
            jnp.int32, (tok.shape[0], x_ref.shape[0]), 1) == tok
        xb = jax.lax.dot_general(sel.astype(jnp.float32), x_ref[...],
                                 (((1,), (0,)), ((), ())),
                                 preferred_element_type=jnp.float32)
        h = jax.lax.dot_general(xb, w1_ref[0], (((1,), (1,)), ((), ())),
                                preferred_element_type=jnp.float32)
        h = h + b1_ref[0]
        h = 0.5 * h * (1.0 + jax.lax.erf(h * 0.7071067811865476))
        y = jax.lax.dot_general(h, w2_ref[0], (((1,), (1,)), ((), ())),
                                preferred_element_type=jnp.float32)
        y = y + b2_ref[0]
        mu = jnp.mean(y, axis=1, keepdims=True)
        yc = y - mu
        var = jnp.mean(yc * yc, axis=1, keepdims=True)
        yn = yc * jax.lax.rsqrt(var + 1e-5)
        o_ref[...] = wp_ref[...] * (yn * g_ref[0] + be_ref[0])


def _combine_sc(y_pad, p0, p1, s_tot):
    d = y_pad.shape[1]
    tok_w = s_tot // NW
    nvec = d // 16

    @functools.partial(
        pl.kernel, mesh=_sc_mesh(),
        out_type=jax.ShapeDtypeStruct((s_tot, d), jnp.float32),
        scratch_types=[
            pltpu.VMEM((tok_w,), jnp.int32),
            pltpu.VMEM((tok_w,), jnp.int32),
            pltpu.VMEM((tok_w, d), jnp.float32),
            pltpu.VMEM((tok_w, d), jnp.float32),
            pltpu.SemaphoreType.DMA,
            pltpu.SemaphoreType.DMA,
        ],
    )
    def k(y_hbm, p0_hbm, p1_hbm, out_hbm, i0_v, i1_v, r0_v, r1_v, s0, s1):
        wid = lax.axis_index("s") * 2 + lax.axis_index("c")
        base = wid * tok_w
        pltpu.sync_copy(p0_hbm.at[pl.ds(base, tok_w)], i0_v)
        pltpu.sync_copy(p1_hbm.at[pl.ds(base, tok_w)], i1_v)
        c0 = pltpu.async_copy(y_hbm.at[i0_v], r0_v, s0)
        c1 = pltpu.async_copy(y_hbm.at[i1_v], r1_v, s1)
        c0.wait()
        c1.wait()

        def row(r, _):
            def colv(c, _):
                r0_v[r, pl.ds(c * 16, 16)] = (r0_v[r, pl.ds(c * 16, 16)]
                                              + r1_v[r, pl.ds(c * 16, 16)])
                return 0
            return lax.fori_loop(0, nvec, colv, 0)

        lax.fori_loop(0, tok_w, row, 0)
        pltpu.sync_copy(r0_v, out_hbm.at[pl.ds(base, tok_w)])

    return k(y_pad, p0, p1)


def kernel(x, gate_w, gate_b, W1, b1, W2, b2, ln_g, ln_b):
    Bs, Ss, D = x.shape
    S = Bs * Ss
    F = 2 * S
    NB = F // BLK + N_EXP
    PAD = NB * BLK
    xf = x.reshape(S, D)

    eids2, nact2, tok2, w2, p0, p1 = pl.pallas_call(
        _route_body,
        out_shape=[
            jax.ShapeDtypeStruct((1, NB), jnp.int32),
            jax.ShapeDtypeStruct((1, 1), jnp.int32),
            jax.ShapeDtypeStruct((NB, BLK), jnp.int32),
            jax.ShapeDtypeStruct((NB, BLK), jnp.float32),
            jax.ShapeDtypeStruct((S, 1), jnp.int32),
            jax.ShapeDtypeStruct((S, 1), jnp.int32),
        ],
    )(xf, gate_w, gate_b.reshape(1, N_EXP))
    block_eids = eids2.reshape(NB)
    nactive = nact2.reshape(1)
    src_tok = tok2.reshape(PAD)
    w_pad = w2.reshape(PAD)

    y_pad = pl.pallas_call(
        _ffn_body,
        grid_spec=pltpu.PrefetchScalarGridSpec(
            num_scalar_prefetch=2,
            grid=(NB,),
            in_specs=[
                pl.BlockSpec((S, D), lambda b, eids, na: (0, 0)),
                pl.BlockSpec((BLK, 1), lambda b, eids, na: (b, 0)),
                pl.BlockSpec((1, D_FF, D),
                             lambda b, eids, na: (eids[b], 0, 0)),
                pl.BlockSpec((1, 1, D_FF),
                             lambda b, eids, na: (eids[b], 0, 0)),
                pl.BlockSpec((1, D, D_FF),
                             lambda b, eids, na: (eids[b], 0, 0)),
                pl.BlockSpec((1, 1, D), lambda b, eids, na: (eids[b], 0, 0)),
                pl.BlockSpec((1, 1, D), lambda b, eids, na: (eids[b], 0, 0)),
                pl.BlockSpec((1, 1, D), lambda b, eids, na: (eids[b], 0, 0)),
                pl.BlockSpec((BLK, 1), lambda b, eids, na: (b, 0)),
            ],
            out_specs=pl.BlockSpec(
                (BLK, D), lambda b, eids, na: (jnp.minimum(b, na[0] - 1), 0)),
        ),
        out_shape=jax.ShapeDtypeStruct((PAD, D), jnp.float32),
        compiler_params=pltpu.CompilerParams(
            dimension_semantics=("arbitrary",)),
    )(block_eids, nactive, xf, src_tok.reshape(PAD, 1), W1,
      b1.reshape(N_EXP, 1, D_FF), W2,
      b2.reshape(N_EXP, 1, D), ln_g.reshape(N_EXP, 1, D),
      ln_b.reshape(N_EXP, 1, D), w_pad.reshape(PAD, 1))

    out = _combine_sc(y_pad, p0.reshape(S), p1.reshape(S), S)
    return out.reshape(Bs, Ss, D)


# speedup vs baseline: 1.0254x; 1.0254x over previous
import functools

import jax
import jax.numpy as jnp
from jax import lax
from jax.experimental import pallas as pl
from jax.experimental.pallas import tpu as pltpu
from jax.experimental.pallas import tpu_sc as plsc

D_MODEL = 768
D_FF = 2048
N_EXP = 64
BLK = 128
NW = 32

def _sc_mesh():
    return plsc.VectorSubcoreMesh(core_axis_name="c", subcore_axis_name="s")


def _route_body(x_ref, gw_ref, gb_ref, eids_ref, nact_ref, tok2_ref, w2_ref,
                p0_ref, p1_ref):
    S = x_ref.shape[0]
    NB = 2 * S // BLK + N_EXP
    CH = 128
    NCH = 2 * S // CH
    HALF = S // CH

    s = jax.lax.dot_general(x_ref[...], gw_ref[...], (((1,), (1,)), ((), ())),
                            preferred_element_type=jnp.float32)
    s = s + gb_ref[...]
    col = jax.lax.broadcasted_iota(jnp.int32, s.shape, 1)
    m0 = jnp.max(s, axis=1, keepdims=True)
    i0 = jnp.min(jnp.where(s == m0, col, N_EXP), axis=1, keepdims=True)
    s1 = jnp.where(col == i0, -jnp.inf, s)
    m1 = jnp.max(s1, axis=1, keepdims=True)
    i1 = jnp.min(jnp.where(s1 == m1, col, N_EXP), axis=1, keepdims=True)
    z = jnp.exp(m1 - m0)
    w0 = 1.0 / (1.0 + z)
    w1 = z / (1.0 + z)

    e64 = jax.lax.broadcasted_iota(jnp.int32, (CH, N_EXP), 1)
    lt = (jax.lax.broadcasted_iota(jnp.int32, (CH, CH), 1)
          < jax.lax.broadcasted_iota(jnp.int32, (CH, CH), 0)).astype(
              jnp.float32)

    def pair_chunk(c):
        if c < HALF:
            return (jax.lax.slice(i0, (c * CH, 0), ((c + 1) * CH, 1)),
                    jax.lax.slice(w0, (c * CH, 0), ((c + 1) * CH, 1)))
        cc = c - HALF
        return (jax.lax.slice(i1, (cc * CH, 0), ((cc + 1) * CH, 1)),
                jax.lax.slice(w1, (cc * CH, 0), ((cc + 1) * CH, 1)))

    running = jnp.zeros((1, N_EXP), jnp.float32)
    ranks = []
    for c in range(NCH):
        ec, _ = pair_chunk(c)
        oh = (ec == e64).astype(jnp.float32)
        base = jnp.sum(running * oh, axis=1, keepdims=True)
        rwithin = jax.lax.dot_general(lt, oh, (((1,), (0,)), ((), ())),
                                      preferred_element_type=jnp.float32)
        rank_in = jnp.sum(rwithin * oh, axis=1, keepdims=True)
        ranks.append(base + rank_in)
        running = running + jnp.sum(oh, axis=0, keepdims=True)
    counts = running

    m_incl = (jax.lax.broadcasted_iota(jnp.int32, (N_EXP, N_EXP), 0)
              <= jax.lax.broadcasted_iota(jnp.int32, (N_EXP, N_EXP), 1)
              ).astype(jnp.float32)
    nblk = jnp.floor((counts + (BLK - 1)) * (1.0 / BLK))
    cumb = jax.lax.dot_general(nblk, m_incl, (((1,), (0,)), ((), ())),
                               preferred_element_type=jnp.float32)
    padded_off = (cumb - nblk) * BLK
    nact_ref[...] = jax.lax.slice(cumb, (0, N_EXP - 1),
                                  (1, N_EXP)).astype(jnp.int32)

    ident = (jax.lax.broadcasted_iota(jnp.int32, (N_EXP, N_EXP), 0)
             == jax.lax.broadcasted_iota(jnp.int32, (N_EXP, N_EXP), 1)
             ).astype(jnp.float32)
    cumb_col = jax.lax.dot_general(ident, cumb, (((1,), (1,)), ((), ())),
                                   preferred_element_type=jnp.float32)
    b_row = jax.lax.broadcasted_iota(jnp.int32, (1, NB), 1).astype(jnp.float32)
    eids = jnp.sum((cumb_col <= b_row).astype(jnp.float32), axis=0,
                   keepdims=True)
    e_iota = jax.lax.broadcasted_iota(jnp.int32, (1, N_EXP),
                                      1).astype(jnp.float32)
    last_e = jnp.max(jnp.where(counts > 0.0, e_iota, 0.0))
    eids_ref[...] = jnp.minimum(eids, last_e).astype(jnp.int32)

    sub128 = jax.lax.broadcasted_iota(jnp.int32, (CH, 1),
                                      0).astype(jnp.float32)
    b_cols = jax.lax.broadcasted_iota(jnp.int32, (CH, NB), 1)
    r_cols = jax.lax.broadcasted_iota(jnp.int32, (CH, CH), 1)
    acc_t = jnp.zeros((NB, BLK), jnp.float32)
    acc_w = jnp.zeros((NB, BLK), jnp.float32)
    for c in range(NCH):
        ec, wc = pair_chunk(c)
        oh = (ec == e64).astype(jnp.float32)
        pos = jnp.sum(padded_off * oh, axis=1, keepdims=True) + ranks[c]
        posi = pos.astype(jnp.int32)
        if c < HALF:
            p0_ref[pl.ds(c * CH, CH), :] = posi
        else:
            p1_ref[pl.ds((c - HALF) * CH, CH), :] = posi
        pos_b = jnp.floor(pos * (1.0 / BLK))
        pos_r = pos - pos_b * BLK
        rsel = (pos_b.astype(jnp.int32) == b_cols).astype(jnp.float32)
        csel = (pos_r.astype(jnp.int32) == r_cols).astype(jnp.float32)
        tokv = (c % HALF) * CH + sub128
        acc_t = acc_t + jax.lax.dot_general(
            rsel * tokv, csel, (((0,), (0,)), ((), ())),
            precision=jax.lax.Precision.HIGHEST,
            preferred_element_type=jnp.float32)
        acc_w = acc_w + jax.lax.dot_general(
            rsel * wc, csel, (((0,), (0,)), ((), ())),
            precision=jax.lax.Precision.HIGHEST,
            preferred_element_type=jnp.float32)
    tok2_ref[...] = acc_t.astype(jnp.int32)
    w2_ref[...] = acc_w


def _ffn_body(eids_ref, nact_ref, x_ref, tok_ref, w1_ref, b1_ref, w2_ref,
              b2_ref, g_ref, be_ref, wp_ref, o_ref):
    @pl.when(pl.program_id(0) < nact_ref[0])
    def _():
        tok = tok_ref[...]
        sel = jax.lax.broadcasted_iota(
            jnp.int32, (tok.shape[0], x_ref.shape[0]), 1) == tok
        xb = jax.lax.dot_general(sel.astype(jnp.float32), x_ref[...],
                                 (((1,), (0,)), ((), ())),
                                 preferred_element_type=jnp.float32)
        h = jax.lax.dot_general(xb, w1_ref[0], (((1,), (1,)), ((), ())),
                                preferred_element_type=jnp.float32)
        h = h + b1_ref[0]
        h = 0.5 * h * (1.0 + jax.lax.erf(h * 0.7071067811865476))
        y = jax.lax.dot_general(h, w2_ref[0], (((1,), (1,)), ((), ())),
                                preferred_element_type=jnp.float32)
        y = y + b2_ref[0]
        mu = jnp.mean(y, axis=1, keepdims=True)
        yc = y - mu
        var = jnp.mean(yc * yc, axis=1, keepdims=True)
        yn = yc * jax.lax.rsqrt(var + 1e-5)
        o_ref[...] = wp_ref[...] * (yn * g_ref[0] + be_ref[0])


def _combine_sc(y_pad, p0, p1, s_tot):
    d = y_pad.shape[1]
    tok_w = s_tot // NW
    nvec = d // 16

    @functools.partial(
        pl.kernel, mesh=_sc_mesh(),
        out_type=jax.ShapeDtypeStruct((s_tot, d), jnp.float32),
        scratch_types=[
            pltpu.VMEM((tok_w,), jnp.int32),
            pltpu.VMEM((tok_w,), jnp.int32),
            pltpu.VMEM((tok_w, d), jnp.float32),
            pltpu.VMEM((tok_w, d), jnp.float32),
            pltpu.SemaphoreType.DMA,
            pltpu.SemaphoreType.DMA,
        ],
    )
    def k(y_hbm, p0_hbm, p1_hbm, out_hbm, i0_v, i1_v, r0_v, r1_v, s0, s1):
        wid = lax.axis_index("s") * 2 + lax.axis_index("c")
        base = wid * tok_w
        pltpu.sync_copy(p0_hbm.at[pl.ds(base, tok_w)], i0_v)
        pltpu.sync_copy(p1_hbm.at[pl.ds(base, tok_w)], i1_v)
        c0 = pltpu.async_copy(y_hbm.at[i0_v], r0_v, s0)
        c1 = pltpu.async_copy(y_hbm.at[i1_v], r1_v, s1)
        c0.wait()
        c1.wait()

        def row(r, _):
            def colv(c, _):
                r0_v[r, pl.ds(c * 16, 16)] = (r0_v[r, pl.ds(c * 16, 16)]
                                              + r1_v[r, pl.ds(c * 16, 16)])
                return 0
            return lax.fori_loop(0, nvec, colv, 0)

        lax.fori_loop(0, tok_w, row, 0)
        pltpu.sync_copy(r0_v, out_hbm.at[pl.ds(base, tok_w)])

    return k(y_pad, p0, p1)


def kernel(x, gate_w, gate_b, W1, b1, W2, b2, ln_g, ln_b):
    Bs, Ss, D = x.shape
    S = Bs * Ss
    F = 2 * S
    NB = F // BLK + N_EXP
    PAD = NB * BLK
    xf = x.reshape(S, D)

    eids2, nact2, tok2, w2, p0, p1 = pl.pallas_call(
        _route_body,
        out_shape=[
            jax.ShapeDtypeStruct((1, NB), jnp.int32),
            jax.ShapeDtypeStruct((1, 1), jnp.int32),
            jax.ShapeDtypeStruct((NB, BLK), jnp.int32),
            jax.ShapeDtypeStruct((NB, BLK), jnp.float32),
            jax.ShapeDtypeStruct((S, 1), jnp.int32),
            jax.ShapeDtypeStruct((S, 1), jnp.int32),
        ],
    )(xf, gate_w, gate_b.reshape(1, N_EXP))
    block_eids = eids2.reshape(NB)
    nactive = nact2.reshape(1)
    src_tok = tok2.reshape(PAD)
    w_pad = w2.reshape(PAD)

    y_pad = pl.pallas_call(
        _ffn_body,
        grid_spec=pltpu.PrefetchScalarGridSpec(
            num_scalar_prefetch=2,
            grid=(NB,),
            in_specs=[
                pl.BlockSpec((S, D), lambda b, eids, na: (0, 0)),
                pl.BlockSpec((BLK, 1), lambda b, eids, na: (b, 0)),
                pl.BlockSpec((1, D_FF, D),
                             lambda b, eids, na: (eids[b], 0, 0)),
                pl.BlockSpec((1, 1, D_FF),
                             lambda b, eids, na: (eids[b], 0, 0)),
                pl.BlockSpec((1, D, D_FF),
                             lambda b, eids, na: (eids[b], 0, 0)),
                pl.BlockSpec((1, 1, D), lambda b, eids, na: (eids[b], 0, 0)),
                pl.BlockSpec((1, 1, D), lambda b, eids, na: (eids[b], 0, 0)),
                pl.BlockSpec((1, 1, D), lambda b, eids, na: (eids[b], 0, 0)),
                pl.BlockSpec((BLK, 1), lambda b, eids, na: (b, 0)),
            ],
            out_specs=pl.BlockSpec(
                (BLK, D), lambda b, eids, na: (jnp.minimum(b, na[0] - 1), 0)),
        ),
        out_shape=jax.ShapeDtypeStruct((PAD, D), jnp.float32),
        compiler_params=pltpu.CompilerParams(
            dimension_semantics=("arbitrary",)),
    )(block_eids, nactive, xf, src_tok.reshape(PAD, 1), W1,
      b1.reshape(N_EXP, 1, D_FF), W2,
      b2.reshape(N_EXP, 1, D), ln_g.reshape(N_EXP, 1, D),
      ln_b.reshape(N_EXP, 1, D), w_pad.reshape(PAD, 1))

    out = _combine_sc(y_pad, p0.reshape(S), p1.reshape(S), S)
    return out.reshape(Bs, Ss, D)
